# Initial kernel scaffold; baseline (speedup 1.0000x reference)
import jax
import jax.numpy as jnp
from jax.experimental import pallas as pl


def kernel(q_cam, t_cam, q_rig, t_rig, camera_index, rig_index):
    raise NotImplementedError("write your pallas kernel here")


# speedup vs baseline: 19.1159x; 19.1159x over previous
import functools

import jax
import jax.numpy as jnp
from jax import lax
from jax.experimental import pallas as pl
from jax.experimental.pallas import tpu as pltpu
from jax.experimental.pallas import tpu_sc as plsc

_LANES = 16


def _full(val, dtype=jnp.int32):
    return jnp.full((_LANES,), val, dtype)


@functools.cache
def _make_sc_compose(B, F, NCAM):
    try:
        info = plsc.get_sparse_core_info()
        NC, NS = info.num_cores, info.num_subcores
    except Exception:
        NC, NS = 2, 16
    NW = NC * NS
    BPW = B // NW
    CH = 2048
    if BPW % CH:
        CH = BPW
    G = CH // _LANES
    NCHUNK = BPW // CH

    mesh = plsc.VectorSubcoreMesh(core_axis_name="c", subcore_axis_name="s")

    @functools.partial(
        pl.kernel,
        mesh=mesh,
        compiler_params=pltpu.CompilerParams(
            needs_layout_passes=False, use_tc_tiling_on_sc=False),
        out_type=jax.ShapeDtypeStruct((B, 16), jnp.float32),
        scratch_types=[
            pltpu.VMEM((NCAM, 8), jnp.float32),
            pltpu.VMEM((CH,), jnp.int32),
            pltpu.VMEM((CH,), jnp.int32),
            pltpu.VMEM((CH, 8), jnp.float32),
            pltpu.VMEM((CH, 16), jnp.float32),
            pltpu.SemaphoreType.DMA,
        ],
    )
    def sc_fn(cam_hbm, rig_hbm, ci_hbm, ri_hbm, out_hbm,
              cam_v, ci_v, ri_v, rows_v, out_v, sem):
        cid = lax.axis_index("c")
        sid = lax.axis_index("s")
        wid = sid * NC + cid

        pltpu.sync_copy(cam_hbm, cam_v)

        lanes = lax.iota(jnp.int32, _LANES)
        zero = jnp.zeros((_LANES,), jnp.float32)
        one = jnp.ones((_LANES,), jnp.float32)

        def init_g(g, carry):
            rid = g * _LANES + lanes
            plsc.store_scatter(out_v, [rid, _full(12)], zero)
            plsc.store_scatter(out_v, [rid, _full(13)], zero)
            plsc.store_scatter(out_v, [rid, _full(14)], zero)
            plsc.store_scatter(out_v, [rid, _full(15)], one)
            return carry

        lax.fori_loop(0, G, init_g, 0)

        def compute_g(g, carry):
            base = g * _LANES
            rid = base + lanes
            ci16 = ci_v[pl.ds(base, _LANES)]

            def camg(col):
                return plsc.load_gather(cam_v, [ci16, _full(col)])

            def rigg(col):
                return plsc.load_gather(rows_v, [rid, _full(col)])

            cw, cx, cy, cz = camg(0), camg(1), camg(2), camg(3)
            tcx, tcy, tcz = camg(4), camg(5), camg(6)
            rw, rx, ry, rz = rigg(0), rigg(1), rigg(2), rigg(3)
            tx, ty, tz = rigg(4), rigg(5), rigg(6)

            w = cw * rw - cx * rx - cy * ry - cz * rz
            x = cw * rx + cx * rw + cy * rz - cz * ry
            y = cw * ry - cx * rz + cy * rw + cz * rx
            z = cw * rz + cx * ry - cy * rx + cz * rw

            x2, y2, z2 = x + x, y + y, z + z
            xx, yy, zz = x * x2, y * y2, z * z2
            xy, xz, yz = x * y2, x * z2, y * z2
            wx, wy, wz = w * x2, w * y2, w * z2
            r00 = 1.0 - (yy + zz)
            r01 = xy - wz
            r02 = xz + wy
            r10 = xy + wz
            r11 = 1.0 - (xx + zz)
            r12 = yz - wx
            r20 = xz - wy
            r21 = yz + wx
            r22 = 1.0 - (xx + yy)

            ux = cy * tz - cz * ty
            uy = cz * tx - cx * tz
            uz = cx * ty - cy * tx
            vx = ux + cw * tx
            vy = uy + cw * ty
            vz = uz + cw * tz
            px = cy * vz - cz * vy
            py = cz * vx - cx * vz
            pz = cx * vy - cy * vx
            ttx = tx + (px + px) + tcx
            tty = ty + (py + py) + tcy
            ttz = tz + (pz + pz) + tcz

            outs = (r00, r01, r02, ttx,
                    r10, r11, r12, tty,
                    r20, r21, r22, ttz)
            for col, val in enumerate(outs):
                plsc.store_scatter(out_v, [rid, _full(col)], val)
            return carry

        for k in range(NCHUNK):
            base = wid * BPW + k * CH
            pltpu.sync_copy(ri_hbm.at[pl.ds(base, CH)], ri_v)
            pltpu.async_copy(rig_hbm.at[ri_v], rows_v, sem).wait()
            pltpu.sync_copy(ci_hbm.at[pl.ds(base, CH)], ci_v)
            lax.fori_loop(0, G, compute_g, 0)
            pltpu.sync_copy(out_v, out_hbm.at[pl.ds(base, CH)])

    return sc_fn


def kernel(q_cam, t_cam, q_rig, t_rig, camera_index, rig_index):
    B = camera_index.shape[0]
    F = q_rig.shape[0]
    NCAM = q_cam.shape[0]
    rig_packed = jnp.concatenate(
        [q_rig.astype(jnp.float32), t_rig.astype(jnp.float32),
         jnp.zeros((F, 1), jnp.float32)], axis=1)
    cam_packed = jnp.concatenate(
        [q_cam.astype(jnp.float32), t_cam.astype(jnp.float32),
         jnp.zeros((NCAM, 1), jnp.float32)], axis=1)
    ci = camera_index.astype(jnp.int32)
    ri = rig_index.astype(jnp.int32)
    out = _make_sc_compose(B, F, NCAM)(cam_packed, rig_packed, ci, ri)
    return out.reshape(B, 4, 4)


# speedup vs baseline: 40.2536x; 2.1058x over previous
import functools

import jax
import jax.numpy as jnp
from jax import lax
from jax.experimental import pallas as pl
from jax.experimental.pallas import tpu as pltpu
from jax.experimental.pallas import tpu_sc as plsc

_LANES = 16


def _full(val, dtype=jnp.int32):
    return jnp.full((_LANES,), val, dtype)


@functools.cache
def _make_sc_compose(B, F, NCAM):
    try:
        info = plsc.get_sparse_core_info()
        NC, NS = info.num_cores, info.num_subcores
    except Exception:
        NC, NS = 2, 16
    NW = NC * NS
    BPW = B // NW
    CH = 2048
    if BPW % CH:
        CH = BPW
    G = CH // _LANES
    NCHUNK = BPW // CH

    mesh = plsc.VectorSubcoreMesh(core_axis_name="c", subcore_axis_name="s")

    @functools.partial(
        pl.kernel,
        mesh=mesh,
        compiler_params=pltpu.CompilerParams(
            needs_layout_passes=False, use_tc_tiling_on_sc=False),
        out_type=jax.ShapeDtypeStruct((16 * B,), jnp.float32),
        scratch_types=[
            pltpu.VMEM((NCAM, 8), jnp.float32),
            pltpu.VMEM((CH,), jnp.int32),
            pltpu.VMEM((CH,), jnp.int32),
            pltpu.VMEM((CH, 8), jnp.float32),
            pltpu.VMEM((16 * CH,), jnp.float32),
            pltpu.SemaphoreType.DMA,
        ],
    )
    def sc_fn(cam_hbm, rig_hbm, ci_hbm, ri_hbm, out_hbm,
              cam_v, ci_v, ri_v, rows_v, out_v, sem):
        cid = lax.axis_index("c")
        sid = lax.axis_index("s")
        wid = sid * NC + cid

        pltpu.sync_copy(cam_hbm, cam_v)

        lanes = lax.iota(jnp.int32, _LANES)
        zero = jnp.zeros((_LANES,), jnp.float32)
        one = jnp.ones((_LANES,), jnp.float32)

        def init_g(g, carry):
            base = g * _LANES
            common = ((base >> 7) << 9) + (base & 127)
            off = 3 * (4 * CH) + common
            out_v[pl.ds(off, _LANES)] = zero
            out_v[pl.ds(off + 128, _LANES)] = zero
            out_v[pl.ds(off + 256, _LANES)] = zero
            out_v[pl.ds(off + 384, _LANES)] = one
            return carry

        lax.fori_loop(0, G, init_g, 0)

        def compute_g(g, carry):
            base = g * _LANES
            rid = base + lanes
            ci16 = ci_v[pl.ds(base, _LANES)]

            def camg(col):
                return plsc.load_gather(cam_v, [ci16, _full(col)])

            cw, cx, cy, cz = camg(0), camg(1), camg(2), camg(3)
            tcx, tcy, tcz = camg(4), camg(5), camg(6)

            def rigg(col):
                return plsc.load_gather(rows_v, [rid, _full(col)])

            rw, rx, ry, rz = rigg(0), rigg(1), rigg(2), rigg(3)
            tx, ty, tz = rigg(4), rigg(5), rigg(6)

            w = cw * rw - cx * rx - cy * ry - cz * rz
            x = cw * rx + cx * rw + cy * rz - cz * ry
            y = cw * ry - cx * rz + cy * rw + cz * rx
            z = cw * rz + cx * ry - cy * rx + cz * rw

            x2, y2, z2 = x + x, y + y, z + z
            xx, yy, zz = x * x2, y * y2, z * z2
            xy, xz, yz = x * y2, x * z2, y * z2
            wx, wy, wz = w * x2, w * y2, w * z2
            r00 = 1.0 - (yy + zz)
            r01 = xy - wz
            r02 = xz + wy
            r10 = xy + wz
            r11 = 1.0 - (xx + zz)
            r12 = yz - wx
            r20 = xz - wy
            r21 = yz + wx
            r22 = 1.0 - (xx + yy)

            ux = cy * tz - cz * ty
            uy = cz * tx - cx * tz
            uz = cx * ty - cy * tx
            vx = ux + cw * tx
            vy = uy + cw * ty
            vz = uz + cw * tz
            px = cy * vz - cz * vy
            py = cz * vx - cx * vz
            pz = cx * vy - cy * vx
            ttx = tx + (px + px) + tcx
            tty = ty + (py + py) + tcy
            ttz = tz + (pz + pz) + tcz

            outs = (r00, r01, r02, ttx,
                    r10, r11, r12, tty,
                    r20, r21, r22, ttz)
            common = ((base >> 7) << 9) + (base & 127)
            for col, val in enumerate(outs):
                r, c = divmod(col, 4)
                out_v[pl.ds(r * (4 * CH) + common + c * 128, _LANES)] = val
            return carry

        for k in range(NCHUNK):
            base = wid * BPW + k * CH
            pltpu.sync_copy(ri_hbm.at[pl.ds(base, CH)], ri_v)
            cg = pltpu.async_copy(rig_hbm.at[ri_v], rows_v, sem)
            pltpu.sync_copy(ci_hbm.at[pl.ds(base, CH)], ci_v)
            cg.wait()
            lax.fori_loop(0, G, compute_g, 0)
            for r in range(4):
                pltpu.sync_copy(
                    out_v.at[pl.ds(r * (4 * CH), 4 * CH)],
                    out_hbm.at[pl.ds(r * (4 * B) + base * 4, 4 * CH)])

    return sc_fn


def kernel(q_cam, t_cam, q_rig, t_rig, camera_index, rig_index):
    B = camera_index.shape[0]
    F = q_rig.shape[0]
    NCAM = q_cam.shape[0]
    rig_packed = jnp.concatenate(
        [q_rig.astype(jnp.float32), t_rig.astype(jnp.float32),
         jnp.zeros((F, 1), jnp.float32)], axis=1)
    cam_packed = jnp.concatenate(
        [q_cam.astype(jnp.float32), t_cam.astype(jnp.float32),
         jnp.zeros((NCAM, 1), jnp.float32)], axis=1)
    ci = camera_index.astype(jnp.int32)
    ri = rig_index.astype(jnp.int32)
    out = _make_sc_compose(B, F, NCAM)(cam_packed, rig_packed, ci, ri)
    return (out.reshape(4, B // 128, 4, 128)
            .transpose(1, 3, 0, 2)
            .reshape(B, 4, 4))


# speedup vs baseline: 55.4788x; 1.3782x over previous
import functools

import jax
import jax.numpy as jnp
from jax import lax
from jax.experimental import pallas as pl
from jax.experimental.pallas import tpu as pltpu
from jax.experimental.pallas import tpu_sc as plsc

_LANES = 16


def _full(val, dtype=jnp.int32):
    return jnp.full((_LANES,), val, dtype)


@functools.cache
def _make_sc_compose(B, F, NCAM):
    try:
        info = plsc.get_sparse_core_info()
        NC, NS = info.num_cores, info.num_subcores
    except Exception:
        NC, NS = 2, 16
    NW = NC * NS
    BPW = B // NW
    CH = 2048
    if BPW % CH:
        CH = BPW
    G = CH // _LANES
    NCHUNK = BPW // CH

    BLK_PER_TILE = 52
    NBLK = NS * BLK_PER_TILE
    FP = NBLK * 128
    PACK_IT = 4
    PACK_BLKS = BLK_PER_TILE // PACK_IT
    PACK_FR = PACK_BLKS * 128

    mesh = plsc.VectorSubcoreMesh(core_axis_name="c", subcore_axis_name="s")

    @functools.partial(
        pl.kernel,
        mesh=mesh,
        compiler_params=pltpu.CompilerParams(
            needs_layout_passes=False, use_tc_tiling_on_sc=False),
        out_type=(jax.ShapeDtypeStruct((16 * B,), jnp.float32),
                  jax.ShapeDtypeStruct((2 * FP, 8), jnp.float32)),
        scratch_types=[
            pltpu.VMEM((NCAM, 8), jnp.float32),
            pltpu.VMEM((CH,), jnp.int32),
            pltpu.VMEM((CH,), jnp.int32),
            pltpu.VMEM((CH, 8), jnp.float32),
            pltpu.VMEM((16 * CH,), jnp.float32),
            pltpu.VMEM((8 * PACK_BLKS, 128), jnp.float32),
            pltpu.VMEM((PACK_FR, 8), jnp.float32),
            pltpu.SemaphoreType.DMA,
        ],
    )
    def sc_fn(cam_hbm, view_hbm, ci_hbm, ri_hbm, out_hbm, scr_hbm,
              cam_v, ci_v, ri_v, rows_v, out_v, blk_v, pack_v, sem):
        cid = lax.axis_index("c")
        sid = lax.axis_index("s")
        wid = sid * NC + cid

        pltpu.sync_copy(cam_hbm, cam_v)

        lanes = lax.iota(jnp.int32, _LANES)
        zero = jnp.zeros((_LANES,), jnp.float32)
        one = jnp.ones((_LANES,), jnp.float32)

        def pack_g(gg, carry):
            bb = gg >> 3
            goff = (gg & 7) * _LANES
            rows = bb * 128 + goff + lanes
            for c in range(8):
                vec = plsc.load_gather(blk_v, [_full(c) + bb * 8,
                                               goff + lanes])
                plsc.store_scatter(pack_v, [rows, _full(c)], vec)
            return carry

        for j in range(PACK_IT):
            iter_start = (sid * BLK_PER_TILE + j * PACK_BLKS)
            pltpu.sync_copy(view_hbm.at[pl.ds(iter_start * 8, 8 * PACK_BLKS)],
                            blk_v)
            lax.fori_loop(0, 8 * PACK_BLKS, pack_g, 0)
            pltpu.sync_copy(
                pack_v,
                scr_hbm.at[pl.ds(cid * FP + iter_start * 128, PACK_FR)])
        plsc.subcore_barrier()

        def init_g(g, carry):
            base = g * _LANES
            common = ((base >> 7) << 9) + (base & 127)
            off = 3 * (4 * CH) + common
            out_v[pl.ds(off, _LANES)] = zero
            out_v[pl.ds(off + 128, _LANES)] = zero
            out_v[pl.ds(off + 256, _LANES)] = zero
            out_v[pl.ds(off + 384, _LANES)] = one
            return carry

        lax.fori_loop(0, G, init_g, 0)

        def compute_g(g, carry):
            base = g * _LANES
            rid = base + lanes
            ci16 = ci_v[pl.ds(base, _LANES)]

            def camg(col):
                return plsc.load_gather(cam_v, [ci16, _full(col)])

            cw, cx, cy, cz = camg(0), camg(1), camg(2), camg(3)
            tcx, tcy, tcz = camg(4), camg(5), camg(6)

            def rigg(col):
                return plsc.load_gather(rows_v, [rid, _full(col)])

            rw, rx, ry, rz = rigg(0), rigg(1), rigg(2), rigg(3)
            tx, ty, tz = rigg(4), rigg(5), rigg(6)

            w = cw * rw - cx * rx - cy * ry - cz * rz
            x = cw * rx + cx * rw + cy * rz - cz * ry
            y = cw * ry - cx * rz + cy * rw + cz * rx
            z = cw * rz + cx * ry - cy * rx + cz * rw

            x2, y2, z2 = x + x, y + y, z + z
            xx, yy, zz = x * x2, y * y2, z * z2
            xy, xz, yz = x * y2, x * z2, y * z2
            wx, wy, wz = w * x2, w * y2, w * z2
            r00 = 1.0 - (yy + zz)
            r01 = xy - wz
            r02 = xz + wy
            r10 = xy + wz
            r11 = 1.0 - (xx + zz)
            r12 = yz - wx
            r20 = xz - wy
            r21 = yz + wx
            r22 = 1.0 - (xx + yy)

            ux = cy * tz - cz * ty
            uy = cz * tx - cx * tz
            uz = cx * ty - cy * tx
            vx = ux + cw * tx
            vy = uy + cw * ty
            vz = uz + cw * tz
            px = cy * vz - cz * vy
            py = cz * vx - cx * vz
            pz = cx * vy - cy * vx
            ttx = tx + (px + px) + tcx
            tty = ty + (py + py) + tcy
            ttz = tz + (pz + pz) + tcz

            outs = (r00, r01, r02, ttx,
                    r10, r11, r12, tty,
                    r20, r21, r22, ttz)
            common = ((base >> 7) << 9) + (base & 127)
            for col, val in enumerate(outs):
                r, c = divmod(col, 4)
                out_v[pl.ds(r * (4 * CH) + common + c * 128, _LANES)] = val
            return carry

        def shift_g(g, carry):
            b2 = g * _LANES
            ri_v[pl.ds(b2, _LANES)] = ri_v[pl.ds(b2, _LANES)] + FP
            return carry

        for k in range(NCHUNK):
            base = wid * BPW + k * CH
            pltpu.sync_copy(ri_hbm.at[pl.ds(base, CH)], ri_v)

            @pl.when(cid == 1)
            def _():
                lax.fori_loop(0, G, shift_g, 0)

            cg = pltpu.async_copy(scr_hbm.at[ri_v], rows_v, sem)
            pltpu.sync_copy(ci_hbm.at[pl.ds(base, CH)], ci_v)
            cg.wait()
            lax.fori_loop(0, G, compute_g, 0)
            for r in range(4):
                pltpu.sync_copy(
                    out_v.at[pl.ds(r * (4 * CH), 4 * CH)],
                    out_hbm.at[pl.ds(r * (4 * B) + base * 4, 4 * CH)])

    return sc_fn


def kernel(q_cam, t_cam, q_rig, t_rig, camera_index, rig_index):
    B = camera_index.shape[0]
    F = q_rig.shape[0]
    NCAM = q_cam.shape[0]
    rig_packed = jnp.concatenate(
        [q_rig.astype(jnp.float32), t_rig.astype(jnp.float32),
         jnp.zeros((F, 1), jnp.float32)], axis=1)
    cam_packed = jnp.concatenate(
        [q_cam.astype(jnp.float32), t_cam.astype(jnp.float32),
         jnp.zeros((NCAM, 1), jnp.float32)], axis=1)
    ci = camera_index.astype(jnp.int32)
    ri = rig_index.astype(jnp.int32)
    FP = 16 * 52 * 128
    rig_view = (jnp.pad(rig_packed, ((0, FP - F), (0, 0)))
                .reshape(FP // 128, 128, 8)
                .transpose(0, 2, 1)
                .reshape(FP * 8 // 128, 128))
    out, _ = _make_sc_compose(B, F, NCAM)(cam_packed, rig_view, ci, ri)
    return (out.reshape(4, B // 128, 4, 128)
            .transpose(1, 3, 0, 2)
            .reshape(B, 4, 4))


# speedup vs baseline: 65.8966x; 1.1878x over previous
import functools

import jax
import jax.numpy as jnp
from jax import lax
from jax.experimental import pallas as pl
from jax.experimental.pallas import tpu as pltpu
from jax.experimental.pallas import tpu_sc as plsc

_LANES = 16


def _full(val, dtype=jnp.int32):
    return jnp.full((_LANES,), val, dtype)


@functools.cache
def _make_sc_compose(B, F, NCAM):
    try:
        info = plsc.get_sparse_core_info()
        NC, NS = info.num_cores, info.num_subcores
    except Exception:
        NC, NS = 2, 16
    NW = NC * NS
    BPW = B // NW
    CH = 2048
    if BPW % CH:
        CH = BPW
    G = CH // _LANES
    NCHUNK = BPW // CH

    PACK_BLKS = 10
    PACK_IT = 5
    BLK_PER_TILE = PACK_BLKS * PACK_IT
    NBLK = NS * BLK_PER_TILE
    FP = NBLK * 128
    PACK_FR = PACK_BLKS * 128
    assert FP >= F

    mesh = plsc.VectorSubcoreMesh(core_axis_name="c", subcore_axis_name="s")

    @functools.partial(
        pl.kernel,
        mesh=mesh,
        compiler_params=pltpu.CompilerParams(
            needs_layout_passes=False, use_tc_tiling_on_sc=False),
        out_type=(jax.ShapeDtypeStruct((16 * B,), jnp.float32),
                  jax.ShapeDtypeStruct((2 * FP, 8), jnp.float32)),
        scratch_types=[
            pltpu.VMEM((NCAM, 8), jnp.float32),
            pltpu.VMEM((CH,), jnp.int32),
            pltpu.VMEM((CH,), jnp.int32),
            pltpu.VMEM((CH,), jnp.int32),
            pltpu.VMEM((CH,), jnp.int32),
            pltpu.VMEM((CH, 8), jnp.float32),
            pltpu.VMEM((CH, 8), jnp.float32),
            pltpu.VMEM((16 * CH,), jnp.float32),
            pltpu.VMEM((16 * CH,), jnp.float32),
            pltpu.VMEM((8 * PACK_BLKS, 128), jnp.float32),
            pltpu.VMEM((PACK_FR, 8), jnp.float32),
            pltpu.SemaphoreType.DMA,
            pltpu.SemaphoreType.DMA,
            pltpu.SemaphoreType.DMA,
            pltpu.SemaphoreType.DMA,
            pltpu.SemaphoreType.DMA,
            pltpu.SemaphoreType.DMA,
            pltpu.SemaphoreType.DMA,
            pltpu.SemaphoreType.DMA,
        ],
    )
    def sc_fn(cam_hbm, view_hbm, ci_hbm, ri_hbm, out_hbm, scr_hbm,
              cam_v, ci_a, ci_b, ri_a, ri_b, rows_a, rows_b, out_a, out_b,
              blk_v, pack_v,
              sem_ga, sem_gb, sem_ria, sem_rib, sem_cia, sem_cib,
              sem_oa, sem_ob):
        cid = lax.axis_index("c")
        sid = lax.axis_index("s")
        wid = sid * NC + cid

        ci_v = (ci_a, ci_b)
        ri_v = (ri_a, ri_b)
        rows_v = (rows_a, rows_b)
        out_v = (out_a, out_b)
        sem_g = (sem_ga, sem_gb)
        sem_ri = (sem_ria, sem_rib)
        sem_ci = (sem_cia, sem_cib)
        sem_o = (sem_oa, sem_ob)

        lanes = lax.iota(jnp.int32, _LANES)
        zero = jnp.zeros((_LANES,), jnp.float32)
        one = jnp.ones((_LANES,), jnp.float32)

        pltpu.sync_copy(cam_hbm, cam_v)
        ri0 = pltpu.async_copy(ri_hbm.at[pl.ds(wid * BPW, CH)], ri_v[0],
                               sem_ri[0])
        ci0 = pltpu.async_copy(ci_hbm.at[pl.ds(wid * BPW, CH)], ci_v[0],
                               sem_ci[0])

        def pack_g(gg, carry):
            bb = gg >> 3
            goff = (gg & 7) * _LANES
            rows = bb * 128 + goff + lanes
            for c in range(8):
                vec = blk_v[bb * 8 + c, pl.ds(goff, _LANES)]
                plsc.store_scatter(pack_v, [rows, _full(c)], vec)
            return carry

        for j in range(PACK_IT):
            iter_start = sid * BLK_PER_TILE + j * PACK_BLKS
            pltpu.sync_copy(
                view_hbm.at[pl.ds(iter_start * 8, 8 * PACK_BLKS)],
                blk_v)
            lax.fori_loop(0, 8 * PACK_BLKS, pack_g, 0)
            pltpu.sync_copy(
                pack_v,
                scr_hbm.at[pl.ds(cid * FP + iter_start * 128, PACK_FR)])
        plsc.subcore_barrier()

        def make_init(out_ref):
            def init_g(g, carry):
                base = g * _LANES
                off = 3 * (4 * CH) + ((base >> 7) << 9) + (base & 127)
                out_ref[pl.ds(off, _LANES)] = zero
                out_ref[pl.ds(off + 128, _LANES)] = zero
                out_ref[pl.ds(off + 256, _LANES)] = zero
                out_ref[pl.ds(off + 384, _LANES)] = one
                return carry
            return init_g

        lax.fori_loop(0, G, make_init(out_v[0]), 0)
        lax.fori_loop(0, G, make_init(out_v[1]), 0)

        def make_shift(ri_ref):
            def shift_g(g, carry):
                b2 = g * _LANES
                ri_ref[pl.ds(b2, _LANES)] = ri_ref[pl.ds(b2, _LANES)] + FP
                return carry
            return shift_g

        def make_compute(ci_ref, rows_ref, out_ref):
            def compute_g(g, carry):
                base = g * _LANES
                rid = base + lanes
                ci16 = ci_ref[pl.ds(base, _LANES)]

                def camg(col):
                    return plsc.load_gather(cam_v, [ci16, _full(col)])

                cw, cx, cy, cz = camg(0), camg(1), camg(2), camg(3)
                tcx, tcy, tcz = camg(4), camg(5), camg(6)

                def rigg(col):
                    return plsc.load_gather(rows_ref, [rid, _full(col)])

                rw, rx, ry, rz = rigg(0), rigg(1), rigg(2), rigg(3)
                tx, ty, tz = rigg(4), rigg(5), rigg(6)

                w = cw * rw - cx * rx - cy * ry - cz * rz
                x = cw * rx + cx * rw + cy * rz - cz * ry
                y = cw * ry - cx * rz + cy * rw + cz * rx
                z = cw * rz + cx * ry - cy * rx + cz * rw

                x2, y2, z2 = x + x, y + y, z + z
                xx, yy, zz = x * x2, y * y2, z * z2
                xy, xz, yz = x * y2, x * z2, y * z2
                wx, wy, wz = w * x2, w * y2, w * z2
                r00 = 1.0 - (yy + zz)
                r01 = xy - wz
                r02 = xz + wy
                r10 = xy + wz
                r11 = 1.0 - (xx + zz)
                r12 = yz - wx
                r20 = xz - wy
                r21 = yz + wx
                r22 = 1.0 - (xx + yy)

                ux = cy * tz - cz * ty
                uy = cz * tx - cx * tz
                uz = cx * ty - cy * tx
                vx = ux + cw * tx
                vy = uy + cw * ty
                vz = uz + cw * tz
                px = cy * vz - cz * vy
                py = cz * vx - cx * vz
                pz = cx * vy - cy * vx
                ttx = tx + (px + px) + tcx
                tty = ty + (py + py) + tcy
                ttz = tz + (pz + pz) + tcz

                outs = (r00, r01, r02, ttx,
                        r10, r11, r12, tty,
                        r20, r21, r22, ttz)
                common = ((base >> 7) << 9) + (base & 127)
                for col, val in enumerate(outs):
                    r, c = divmod(col, 4)
                    out_ref[pl.ds(r * (4 * CH) + common + c * 128,
                                  _LANES)] = val
                return carry
            return compute_g

        ri0.wait()

        @pl.when(cid == 1)
        def _():
            lax.fori_loop(0, G, make_shift(ri_v[0]), 0)

        gathers = [None, None]
        gathers[0] = pltpu.async_copy(scr_hbm.at[ri_v[0]], rows_v[0],
                                      sem_g[0])
        ci_copies = [ci0, None]
        out_copies = [None, None]

        for k in range(NCHUNK):
            b = k % 2
            nb = 1 - b
            if k + 1 < NCHUNK:
                nbase = wid * BPW + (k + 1) * CH
                rin = pltpu.async_copy(ri_hbm.at[pl.ds(nbase, CH)],
                                       ri_v[nb], sem_ri[nb])
                ci_copies[nb] = pltpu.async_copy(
                    ci_hbm.at[pl.ds(nbase, CH)], ci_v[nb], sem_ci[nb])
            gathers[b].wait()
            ci_copies[b].wait()
            if k + 1 < NCHUNK:
                rin.wait()

                @pl.when(cid == 1)
                def _():
                    lax.fori_loop(0, G, make_shift(ri_v[nb]), 0)

                gathers[nb] = pltpu.async_copy(scr_hbm.at[ri_v[nb]],
                                               rows_v[nb], sem_g[nb])
            if k >= 2:
                for h in out_copies[b]:
                    h.wait()
            lax.fori_loop(0, G, make_compute(ci_v[b], rows_v[b], out_v[b]), 0)
            base = wid * BPW + k * CH
            out_copies[b] = [
                pltpu.async_copy(
                    out_v[b].at[pl.ds(r * (4 * CH), 4 * CH)],
                    out_hbm.at[pl.ds(r * (4 * B) + base * 4, 4 * CH)],
                    sem_o[b])
                for r in range(4)]
        for b in range(2):
            for h in out_copies[b]:
                h.wait()

    return sc_fn


def kernel(q_cam, t_cam, q_rig, t_rig, camera_index, rig_index):
    B = camera_index.shape[0]
    F = q_rig.shape[0]
    NCAM = q_cam.shape[0]
    rig_packed = jnp.concatenate(
        [q_rig.astype(jnp.float32), t_rig.astype(jnp.float32),
         jnp.zeros((F, 1), jnp.float32)], axis=1)
    cam_packed = jnp.concatenate(
        [q_cam.astype(jnp.float32), t_cam.astype(jnp.float32),
         jnp.zeros((NCAM, 1), jnp.float32)], axis=1)
    ci = camera_index.astype(jnp.int32)
    ri = rig_index.astype(jnp.int32)
    FP = 16 * 50 * 128
    rig_view = (jnp.pad(rig_packed, ((0, FP - F), (0, 0)))
                .reshape(FP // 128, 128, 8)
                .transpose(0, 2, 1)
                .reshape(FP * 8 // 128, 128))
    out, _ = _make_sc_compose(B, F, NCAM)(cam_packed, rig_view, ci, ri)
    return (out.reshape(4, B // 128, 4, 128)
            .transpose(1, 3, 0, 2)
            .reshape(B, 4, 4))


# speedup vs baseline: 72.0942x; 1.0940x over previous
import functools

import jax
import jax.numpy as jnp
from jax import lax
from jax.experimental import pallas as pl
from jax.experimental.pallas import tpu as pltpu
from jax.experimental.pallas import tpu_sc as plsc

_LANES = 16


def _full(val, dtype=jnp.int32):
    return jnp.full((_LANES,), val, dtype)


@functools.cache
def _make_sc_compose(B, F, NCAM):
    try:
        info = plsc.get_sparse_core_info()
        NC, NS = info.num_cores, info.num_subcores
    except Exception:
        NC, NS = 2, 16
    NW = NC * NS
    BPW = B // NW
    CH = 2048
    if BPW % CH:
        CH = BPW
    G = CH // _LANES
    NCHUNK = BPW // CH

    PACK_BLKS = 5
    PACK_IT = 10
    BLK_PER_TILE = PACK_BLKS * PACK_IT
    NBLK = NS * BLK_PER_TILE
    FP = NBLK * 128
    PACK_FR = PACK_BLKS * 128
    assert FP >= F

    mesh = plsc.VectorSubcoreMesh(core_axis_name="c", subcore_axis_name="s")

    @functools.partial(
        pl.kernel,
        mesh=mesh,
        compiler_params=pltpu.CompilerParams(
            needs_layout_passes=False, use_tc_tiling_on_sc=False),
        out_type=(jax.ShapeDtypeStruct((16 * B,), jnp.float32),
                  jax.ShapeDtypeStruct((2 * FP, 8), jnp.float32)),
        scratch_types=[
            pltpu.VMEM((NCAM, 8), jnp.float32),
            pltpu.VMEM((CH,), jnp.int32),
            pltpu.VMEM((CH,), jnp.int32),
            pltpu.VMEM((CH,), jnp.int32),
            pltpu.VMEM((CH,), jnp.int32),
            pltpu.VMEM((CH, 8), jnp.float32),
            pltpu.VMEM((CH, 8), jnp.float32),
            pltpu.VMEM((16 * CH,), jnp.float32),
            pltpu.VMEM((16 * CH,), jnp.float32),
            pltpu.VMEM((8 * PACK_BLKS, 128), jnp.float32),
            pltpu.VMEM((8 * PACK_BLKS, 128), jnp.float32),
            pltpu.VMEM((PACK_FR, 8), jnp.float32),
            pltpu.VMEM((PACK_FR, 8), jnp.float32),
            pltpu.SemaphoreType.DMA,
            pltpu.SemaphoreType.DMA,
            pltpu.SemaphoreType.DMA,
            pltpu.SemaphoreType.DMA,
            pltpu.SemaphoreType.DMA,
            pltpu.SemaphoreType.DMA,
            pltpu.SemaphoreType.DMA,
            pltpu.SemaphoreType.DMA,
            pltpu.SemaphoreType.DMA,
            pltpu.SemaphoreType.DMA,
            pltpu.SemaphoreType.DMA,
            pltpu.SemaphoreType.DMA,
        ],
    )
    def sc_fn(cam_hbm, view_hbm, ci_hbm, ri_hbm, out_hbm, scr_hbm,
              cam_v, ci_a, ci_b, ri_a, ri_b, rows_a, rows_b, out_a, out_b,
              blk_a, blk_b, pack_a, pack_b,
              sem_pia, sem_pib, sem_poa, sem_pob,
              sem_ga, sem_gb, sem_ria, sem_rib, sem_cia, sem_cib,
              sem_oa, sem_ob):
        cid = lax.axis_index("c")
        sid = lax.axis_index("s")
        wid = sid * NC + cid

        ci_v = (ci_a, ci_b)
        ri_v = (ri_a, ri_b)
        rows_v = (rows_a, rows_b)
        out_v = (out_a, out_b)
        sem_g = (sem_ga, sem_gb)
        sem_ri = (sem_ria, sem_rib)
        sem_ci = (sem_cia, sem_cib)
        sem_o = (sem_oa, sem_ob)

        lanes = lax.iota(jnp.int32, _LANES)
        zero = jnp.zeros((_LANES,), jnp.float32)
        one = jnp.ones((_LANES,), jnp.float32)

        pltpu.sync_copy(cam_hbm, cam_v)
        ri0 = pltpu.async_copy(ri_hbm.at[pl.ds(wid * BPW, CH)], ri_v[0],
                               sem_ri[0])
        ci0 = pltpu.async_copy(ci_hbm.at[pl.ds(wid * BPW, CH)], ci_v[0],
                               sem_ci[0])

        blk_v = (blk_a, blk_b)
        pack_v = (pack_a, pack_b)
        sem_pi = (sem_pia, sem_pib)
        sem_po = (sem_poa, sem_pob)

        def make_pack(blk_ref, pack_ref):
            def pack_g(gg, carry):
                bb = gg >> 3
                goff = (gg & 7) * _LANES
                rows = bb * 128 + goff + lanes
                for c in range(8):
                    vec = blk_ref[bb * 8 + c, pl.ds(goff, _LANES)]
                    plsc.store_scatter(pack_ref, [rows, _full(c)], vec)
                return carry
            return pack_g

        def pack_start(j, b):
            iter_start = sid * BLK_PER_TILE + j * PACK_BLKS
            return pltpu.async_copy(
                view_hbm.at[pl.ds(iter_start * 8, 8 * PACK_BLKS)],
                blk_v[b], sem_pi[b])

        pins = [pack_start(0, 0), None]
        pouts = [None, None]
        for j in range(PACK_IT):
            b = j % 2
            nb = 1 - b
            if j + 1 < PACK_IT:
                pins[nb] = pack_start(j + 1, nb)
            pins[b].wait()
            if j >= 2:
                pouts[b].wait()
            lax.fori_loop(0, 8 * PACK_BLKS, make_pack(blk_v[b], pack_v[b]), 0)
            iter_start = sid * BLK_PER_TILE + j * PACK_BLKS
            pouts[b] = pltpu.async_copy(
                pack_v[b],
                scr_hbm.at[pl.ds(cid * FP + iter_start * 128, PACK_FR)],
                sem_po[b])
        pouts[0].wait()
        pouts[1].wait()
        plsc.subcore_barrier()

        def make_init(out_ref):
            def init_g(g, carry):
                base = g * _LANES
                off = 3 * (4 * CH) + ((base >> 7) << 9) + (base & 127)
                out_ref[pl.ds(off, _LANES)] = zero
                out_ref[pl.ds(off + 128, _LANES)] = zero
                out_ref[pl.ds(off + 256, _LANES)] = zero
                out_ref[pl.ds(off + 384, _LANES)] = one
                return carry
            return init_g

        lax.fori_loop(0, G, make_init(out_v[0]), 0)
        lax.fori_loop(0, G, make_init(out_v[1]), 0)

        def make_shift(ri_ref):
            def shift_g(g, carry):
                b2 = g * _LANES
                ri_ref[pl.ds(b2, _LANES)] = ri_ref[pl.ds(b2, _LANES)] + FP
                return carry
            return shift_g

        def make_compute(ci_ref, rows_ref, out_ref):
            def compute_g(g, carry):
                base = g * _LANES
                rid = base + lanes
                ci16 = ci_ref[pl.ds(base, _LANES)]

                def camg(col):
                    return plsc.load_gather(cam_v, [ci16, _full(col)])

                cw, cx, cy, cz = camg(0), camg(1), camg(2), camg(3)
                tcx, tcy, tcz = camg(4), camg(5), camg(6)

                def rigg(col):
                    return plsc.load_gather(rows_ref, [rid, _full(col)])

                rw, rx, ry, rz = rigg(0), rigg(1), rigg(2), rigg(3)
                tx, ty, tz = rigg(4), rigg(5), rigg(6)

                w = cw * rw - cx * rx - cy * ry - cz * rz
                x = cw * rx + cx * rw + cy * rz - cz * ry
                y = cw * ry - cx * rz + cy * rw + cz * rx
                z = cw * rz + cx * ry - cy * rx + cz * rw

                x2, y2, z2 = x + x, y + y, z + z
                xx, yy, zz = x * x2, y * y2, z * z2
                xy, xz, yz = x * y2, x * z2, y * z2
                wx, wy, wz = w * x2, w * y2, w * z2
                r00 = 1.0 - (yy + zz)
                r01 = xy - wz
                r02 = xz + wy
                r10 = xy + wz
                r11 = 1.0 - (xx + zz)
                r12 = yz - wx
                r20 = xz - wy
                r21 = yz + wx
                r22 = 1.0 - (xx + yy)

                ux = cy * tz - cz * ty
                uy = cz * tx - cx * tz
                uz = cx * ty - cy * tx
                vx = ux + cw * tx
                vy = uy + cw * ty
                vz = uz + cw * tz
                px = cy * vz - cz * vy
                py = cz * vx - cx * vz
                pz = cx * vy - cy * vx
                ttx = tx + (px + px) + tcx
                tty = ty + (py + py) + tcy
                ttz = tz + (pz + pz) + tcz

                outs = (r00, r01, r02, ttx,
                        r10, r11, r12, tty,
                        r20, r21, r22, ttz)
                common = ((base >> 7) << 9) + (base & 127)
                for col, val in enumerate(outs):
                    r, c = divmod(col, 4)
                    out_ref[pl.ds(r * (4 * CH) + common + c * 128,
                                  _LANES)] = val
                return carry
            return compute_g

        ri0.wait()

        @pl.when(cid == 1)
        def _():
            lax.fori_loop(0, G, make_shift(ri_v[0]), 0)

        gathers = [None, None]
        gathers[0] = pltpu.async_copy(scr_hbm.at[ri_v[0]], rows_v[0],
                                      sem_g[0])
        ci_copies = [ci0, None]
        out_copies = [None, None]

        for k in range(NCHUNK):
            b = k % 2
            nb = 1 - b
            if k + 1 < NCHUNK:
                nbase = wid * BPW + (k + 1) * CH
                rin = pltpu.async_copy(ri_hbm.at[pl.ds(nbase, CH)],
                                       ri_v[nb], sem_ri[nb])
                ci_copies[nb] = pltpu.async_copy(
                    ci_hbm.at[pl.ds(nbase, CH)], ci_v[nb], sem_ci[nb])
            gathers[b].wait()
            ci_copies[b].wait()
            if k + 1 < NCHUNK:
                rin.wait()

                @pl.when(cid == 1)
                def _():
                    lax.fori_loop(0, G, make_shift(ri_v[nb]), 0)

                gathers[nb] = pltpu.async_copy(scr_hbm.at[ri_v[nb]],
                                               rows_v[nb], sem_g[nb])
            if k >= 2:
                for h in out_copies[b]:
                    h.wait()
            lax.fori_loop(0, G, make_compute(ci_v[b], rows_v[b], out_v[b]), 0)
            base = wid * BPW + k * CH
            out_copies[b] = [
                pltpu.async_copy(
                    out_v[b].at[pl.ds(r * (4 * CH), 4 * CH)],
                    out_hbm.at[pl.ds(r * (4 * B) + base * 4, 4 * CH)],
                    sem_o[b])
                for r in range(4)]
        for b in range(2):
            for h in out_copies[b]:
                h.wait()

    return sc_fn


def kernel(q_cam, t_cam, q_rig, t_rig, camera_index, rig_index):
    B = camera_index.shape[0]
    F = q_rig.shape[0]
    NCAM = q_cam.shape[0]
    FP = 16 * 50 * 128
    cam_packed = jnp.concatenate(
        [q_cam.astype(jnp.float32), t_cam.astype(jnp.float32),
         jnp.zeros((NCAM, 1), jnp.float32)], axis=1)
    ci = camera_index.astype(jnp.int32)
    ri = rig_index.astype(jnp.int32)
    rig_view = (jnp.concatenate(
        [jnp.pad(q_rig.astype(jnp.float32), ((0, FP - F), (0, 0))),
         jnp.pad(t_rig.astype(jnp.float32), ((0, FP - F), (0, 1)))], axis=1)
                .reshape(FP // 128, 128, 8)
                .transpose(0, 2, 1)
                .reshape(FP * 8 // 128, 128))
    out, _ = _make_sc_compose(B, F, NCAM)(cam_packed, rig_view, ci, ri)
    return (out.reshape(4, B // 128, 4, 128)
            .transpose(1, 3, 0, 2)
            .reshape(B, 4, 4))


# speedup vs baseline: 86.0714x; 1.1939x over previous
import functools

import jax
import jax.numpy as jnp
from jax import lax
from jax.experimental import pallas as pl
from jax.experimental.pallas import tpu as pltpu
from jax.experimental.pallas import tpu_sc as plsc

_LANES = 16


def _full(val, dtype=jnp.int32):
    return jnp.full((_LANES,), val, dtype)


@functools.cache
def _make_sc_compose(B, F, NCAM):
    try:
        info = plsc.get_sparse_core_info()
        NC, NS = info.num_cores, info.num_subcores
    except Exception:
        NC, NS = 2, 16
    NW = NC * NS
    BPW = B // NW
    CH = 2048
    if BPW % CH:
        CH = BPW
    G = CH // _LANES
    NCHUNK = BPW // CH

    PACK_BLKS = 5
    PACK_IT = 10
    BLK_PER_TILE = PACK_BLKS * PACK_IT
    NBLK = NS * BLK_PER_TILE
    FP = NBLK * 128
    PACK_FR = PACK_BLKS * 128
    assert FP >= F

    mesh = plsc.VectorSubcoreMesh(core_axis_name="c", subcore_axis_name="s")

    @functools.partial(
        pl.kernel,
        mesh=mesh,
        compiler_params=pltpu.CompilerParams(
            needs_layout_passes=False, use_tc_tiling_on_sc=False),
        out_type=(jax.ShapeDtypeStruct((16 * B,), jnp.float32),
                  jax.ShapeDtypeStruct((2 * FP, 8), jnp.float32)),
        scratch_types=[
            pltpu.VMEM((NCAM, 8), jnp.float32),
            pltpu.VMEM((CH,), jnp.int32),
            pltpu.VMEM((CH,), jnp.int32),
            pltpu.VMEM((CH,), jnp.int32),
            pltpu.VMEM((CH,), jnp.int32),
            pltpu.VMEM((CH, 8), jnp.float32),
            pltpu.VMEM((CH, 8), jnp.float32),
            pltpu.VMEM((16 * CH,), jnp.float32),
            pltpu.VMEM((16 * CH,), jnp.float32),
            pltpu.VMEM((8 * PACK_BLKS, 128), jnp.float32),
            pltpu.VMEM((8 * PACK_BLKS, 128), jnp.float32),
            pltpu.VMEM((PACK_FR, 8), jnp.float32),
            pltpu.VMEM((PACK_FR, 8), jnp.float32),
            pltpu.SemaphoreType.DMA,
            pltpu.SemaphoreType.DMA,
            pltpu.SemaphoreType.DMA,
            pltpu.SemaphoreType.DMA,
            pltpu.SemaphoreType.DMA,
            pltpu.SemaphoreType.DMA,
            pltpu.SemaphoreType.DMA,
            pltpu.SemaphoreType.DMA,
            pltpu.SemaphoreType.DMA,
            pltpu.SemaphoreType.DMA,
            pltpu.SemaphoreType.DMA,
            pltpu.SemaphoreType.DMA,
        ],
    )
    def sc_fn(cam_hbm, view_hbm, ci_hbm, ri_hbm, out_hbm, scr_hbm,
              cam_v, ci_a, ci_b, ri_a, ri_b, rows_a, rows_b, out_a, out_b,
              blk_a, blk_b, pack_a, pack_b,
              sem_pia, sem_pib, sem_poa, sem_pob,
              sem_ga, sem_gb, sem_ria, sem_rib, sem_cia, sem_cib,
              sem_oa, sem_ob):
        cid = lax.axis_index("c")
        sid = lax.axis_index("s")
        wid = sid * NC + cid

        ci_v = (ci_a, ci_b)
        ri_v = (ri_a, ri_b)
        rows_v = (rows_a, rows_b)
        out_v = (out_a, out_b)
        sem_g = (sem_ga, sem_gb)
        sem_ri = (sem_ria, sem_rib)
        sem_ci = (sem_cia, sem_cib)
        sem_o = (sem_oa, sem_ob)

        lanes = lax.iota(jnp.int32, _LANES)
        zero = jnp.zeros((_LANES,), jnp.float32)
        one = jnp.ones((_LANES,), jnp.float32)

        pltpu.sync_copy(cam_hbm, cam_v)
        ri0 = pltpu.async_copy(ri_hbm.at[pl.ds(wid * BPW, CH)], ri_v[0],
                               sem_ri[0])
        ci0 = pltpu.async_copy(ci_hbm.at[pl.ds(wid * BPW, CH)], ci_v[0],
                               sem_ci[0])

        blk_v = (blk_a, blk_b)
        pack_v = (pack_a, pack_b)
        sem_pi = (sem_pia, sem_pib)
        sem_po = (sem_poa, sem_pob)

        def make_pack(blk_ref, pack_ref):
            def pack_g(gg):
                bb = gg >> 3
                goff = (gg & 7) * _LANES
                rows = bb * 128 + goff + lanes
                for c in range(8):
                    vec = blk_ref[bb * 8 + c, pl.ds(goff, _LANES)]
                    plsc.store_scatter(pack_ref, [rows, _full(c)], vec)
            return pack_g

        def pack_start(j, b):
            iter_start = sid * BLK_PER_TILE + j * PACK_BLKS
            return pltpu.async_copy(
                view_hbm.at[pl.ds(iter_start * 8, 8 * PACK_BLKS)],
                blk_v[b], sem_pi[b])

        pins = [pack_start(0, 0), None]
        pouts = [None, None]
        for j in range(PACK_IT):
            b = j % 2
            nb = 1 - b
            if j + 1 < PACK_IT:
                pins[nb] = pack_start(j + 1, nb)
            pins[b].wait()
            if j >= 2:
                pouts[b].wait()
            plsc.parallel_loop(0, 8 * PACK_BLKS)(make_pack(blk_v[b], pack_v[b]))
            iter_start = sid * BLK_PER_TILE + j * PACK_BLKS
            pouts[b] = pltpu.async_copy(
                pack_v[b],
                scr_hbm.at[pl.ds(cid * FP + iter_start * 128, PACK_FR)],
                sem_po[b])
        pouts[0].wait()
        pouts[1].wait()
        plsc.subcore_barrier()

        def make_init(out_ref):
            def init_g(g):
                base = g * _LANES
                off = 3 * (4 * CH) + ((base >> 7) << 9) + (base & 127)
                out_ref[pl.ds(off, _LANES)] = zero
                out_ref[pl.ds(off + 128, _LANES)] = zero
                out_ref[pl.ds(off + 256, _LANES)] = zero
                out_ref[pl.ds(off + 384, _LANES)] = one
            return init_g

        plsc.parallel_loop(0, G)(make_init(out_v[0]))
        plsc.parallel_loop(0, G)(make_init(out_v[1]))

        def make_shift(ri_ref):
            def shift_g(g):
                b2 = g * _LANES
                ri_ref[pl.ds(b2, _LANES)] = ri_ref[pl.ds(b2, _LANES)] + FP
            return shift_g

        def make_compute(ci_ref, rows_ref, out_ref):
            def compute_g(g):
                base = g * _LANES
                rid = base + lanes
                ci16 = ci_ref[pl.ds(base, _LANES)]

                def camg(col):
                    return plsc.load_gather(cam_v, [ci16, _full(col)])

                cw, cx, cy, cz = camg(0), camg(1), camg(2), camg(3)
                tcx, tcy, tcz = camg(4), camg(5), camg(6)

                def rigg(col):
                    return plsc.load_gather(rows_ref, [rid, _full(col)])

                rw, rx, ry, rz = rigg(0), rigg(1), rigg(2), rigg(3)
                tx, ty, tz = rigg(4), rigg(5), rigg(6)

                w = cw * rw - cx * rx - cy * ry - cz * rz
                x = cw * rx + cx * rw + cy * rz - cz * ry
                y = cw * ry - cx * rz + cy * rw + cz * rx
                z = cw * rz + cx * ry - cy * rx + cz * rw

                x2, y2, z2 = x + x, y + y, z + z
                xx, yy, zz = x * x2, y * y2, z * z2
                xy, xz, yz = x * y2, x * z2, y * z2
                wx, wy, wz = w * x2, w * y2, w * z2
                r00 = 1.0 - (yy + zz)
                r01 = xy - wz
                r02 = xz + wy
                r10 = xy + wz
                r11 = 1.0 - (xx + zz)
                r12 = yz - wx
                r20 = xz - wy
                r21 = yz + wx
                r22 = 1.0 - (xx + yy)

                ux = cy * tz - cz * ty
                uy = cz * tx - cx * tz
                uz = cx * ty - cy * tx
                vx = ux + cw * tx
                vy = uy + cw * ty
                vz = uz + cw * tz
                px = cy * vz - cz * vy
                py = cz * vx - cx * vz
                pz = cx * vy - cy * vx
                ttx = tx + (px + px) + tcx
                tty = ty + (py + py) + tcy
                ttz = tz + (pz + pz) + tcz

                outs = (r00, r01, r02, ttx,
                        r10, r11, r12, tty,
                        r20, r21, r22, ttz)
                common = ((base >> 7) << 9) + (base & 127)
                for col, val in enumerate(outs):
                    r, c = divmod(col, 4)
                    out_ref[pl.ds(r * (4 * CH) + common + c * 128,
                                  _LANES)] = val
            return compute_g

        ri0.wait()

        @pl.when(cid == 1)
        def _():
            plsc.parallel_loop(0, G)(make_shift(ri_v[0]))

        gathers = [None, None]
        gathers[0] = pltpu.async_copy(scr_hbm.at[ri_v[0]], rows_v[0],
                                      sem_g[0])
        ci_copies = [ci0, None]
        out_copies = [None, None]

        for k in range(NCHUNK):
            b = k % 2
            nb = 1 - b
            if k + 1 < NCHUNK:
                nbase = wid * BPW + (k + 1) * CH
                rin = pltpu.async_copy(ri_hbm.at[pl.ds(nbase, CH)],
                                       ri_v[nb], sem_ri[nb])
                ci_copies[nb] = pltpu.async_copy(
                    ci_hbm.at[pl.ds(nbase, CH)], ci_v[nb], sem_ci[nb])
            gathers[b].wait()
            ci_copies[b].wait()
            if k + 1 < NCHUNK:
                rin.wait()

                @pl.when(cid == 1)
                def _():
                    plsc.parallel_loop(0, G)(make_shift(ri_v[nb]))

                gathers[nb] = pltpu.async_copy(scr_hbm.at[ri_v[nb]],
                                               rows_v[nb], sem_g[nb])
            if k >= 2:
                for h in out_copies[b]:
                    h.wait()
            plsc.parallel_loop(0, G)(
                make_compute(ci_v[b], rows_v[b], out_v[b]))
            base = wid * BPW + k * CH
            out_copies[b] = [
                pltpu.async_copy(
                    out_v[b].at[pl.ds(r * (4 * CH), 4 * CH)],
                    out_hbm.at[pl.ds(r * (4 * B) + base * 4, 4 * CH)],
                    sem_o[b])
                for r in range(4)]
        for b in range(2):
            for h in out_copies[b]:
                h.wait()

    return sc_fn


def kernel(q_cam, t_cam, q_rig, t_rig, camera_index, rig_index):
    B = camera_index.shape[0]
    F = q_rig.shape[0]
    NCAM = q_cam.shape[0]
    FP = 16 * 50 * 128
    cam_packed = jnp.concatenate(
        [q_cam.astype(jnp.float32), t_cam.astype(jnp.float32),
         jnp.zeros((NCAM, 1), jnp.float32)], axis=1)
    ci = camera_index.astype(jnp.int32)
    ri = rig_index.astype(jnp.int32)
    rig_view = (jnp.concatenate(
        [jnp.pad(q_rig.astype(jnp.float32), ((0, FP - F), (0, 0))),
         jnp.pad(t_rig.astype(jnp.float32), ((0, FP - F), (0, 1)))], axis=1)
                .reshape(FP // 128, 128, 8)
                .transpose(0, 2, 1)
                .reshape(FP * 8 // 128, 128))
    out, _ = _make_sc_compose(B, F, NCAM)(cam_packed, rig_view, ci, ri)
    return (out.reshape(4, B // 128, 4, 128)
            .transpose(1, 3, 0, 2)
            .reshape(B, 4, 4))


# speedup vs baseline: 88.0195x; 1.0226x over previous
import functools

import jax
import jax.numpy as jnp
from jax import lax
from jax.experimental import pallas as pl
from jax.experimental.pallas import tpu as pltpu
from jax.experimental.pallas import tpu_sc as plsc

_LANES = 16


def _full(val, dtype=jnp.int32):
    return jnp.full((_LANES,), val, dtype)


@functools.cache
def _make_sc_compose(B, F, NCAM):
    try:
        info = plsc.get_sparse_core_info()
        NC, NS = info.num_cores, info.num_subcores
    except Exception:
        NC, NS = 2, 16
    NW = NC * NS
    BPW = B // NW
    CH = 2048
    if BPW % CH:
        CH = BPW
    G = CH // _LANES
    NCHUNK = BPW // CH

    PACK_BLKS = 5
    PACK_IT = 10
    BLK_PER_TILE = PACK_BLKS * PACK_IT
    NBLK = NS * BLK_PER_TILE
    FP = NBLK * 128
    PACK_FR = PACK_BLKS * 128
    assert FP >= F

    mesh = plsc.VectorSubcoreMesh(core_axis_name="c", subcore_axis_name="s")

    @functools.partial(
        pl.kernel,
        mesh=mesh,
        compiler_params=pltpu.CompilerParams(
            needs_layout_passes=False, use_tc_tiling_on_sc=False),
        out_type=(jax.ShapeDtypeStruct((16 * B,), jnp.float32),
                  jax.ShapeDtypeStruct((2 * FP, 8), jnp.float32)),
        scratch_types=[
            pltpu.VMEM((NCAM, 8), jnp.float32),
            pltpu.VMEM((CH,), jnp.int32),
            pltpu.VMEM((CH,), jnp.int32),
            pltpu.VMEM((CH,), jnp.int32),
            pltpu.VMEM((CH,), jnp.int32),
            pltpu.VMEM((CH, 8), jnp.float32),
            pltpu.VMEM((CH, 8), jnp.float32),
            pltpu.VMEM((16 * CH,), jnp.float32),
            pltpu.VMEM((16 * CH,), jnp.float32),
            pltpu.VMEM((8 * PACK_BLKS, 128), jnp.float32),
            pltpu.VMEM((8 * PACK_BLKS, 128), jnp.float32),
            pltpu.VMEM((PACK_FR, 8), jnp.float32),
            pltpu.VMEM((PACK_FR, 8), jnp.float32),
            pltpu.SemaphoreType.DMA,
            pltpu.SemaphoreType.DMA,
            pltpu.SemaphoreType.DMA,
            pltpu.SemaphoreType.DMA,
            pltpu.SemaphoreType.DMA,
            pltpu.SemaphoreType.DMA,
            pltpu.SemaphoreType.DMA,
            pltpu.SemaphoreType.DMA,
            pltpu.SemaphoreType.DMA,
            pltpu.SemaphoreType.DMA,
            pltpu.SemaphoreType.DMA,
            pltpu.SemaphoreType.DMA,
        ],
    )
    def sc_fn(cam_hbm, view_hbm, ci_hbm, ri_hbm, out_hbm, scr_hbm,
              cam_v, ci_a, ci_b, ri_a, ri_b, rows_a, rows_b, out_a, out_b,
              blk_a, blk_b, pack_a, pack_b,
              sem_pia, sem_pib, sem_poa, sem_pob,
              sem_ga, sem_gb, sem_ria, sem_rib, sem_cia, sem_cib,
              sem_oa, sem_ob):
        cid = lax.axis_index("c")
        sid = lax.axis_index("s")
        wid = sid * NC + cid

        ci_v = (ci_a, ci_b)
        ri_v = (ri_a, ri_b)
        rows_v = (rows_a, rows_b)
        out_v = (out_a, out_b)
        sem_g = (sem_ga, sem_gb)
        sem_ri = (sem_ria, sem_rib)
        sem_ci = (sem_cia, sem_cib)
        sem_o = (sem_oa, sem_ob)

        lanes = lax.iota(jnp.int32, _LANES)
        zero = jnp.zeros((_LANES,), jnp.float32)
        one = jnp.ones((_LANES,), jnp.float32)

        pltpu.sync_copy(cam_hbm, cam_v)
        ri0 = pltpu.async_copy(ri_hbm.at[pl.ds(wid * BPW, CH)], ri_v[0],
                               sem_ri[0])
        ci0 = pltpu.async_copy(ci_hbm.at[pl.ds(wid * BPW, CH)], ci_v[0],
                               sem_ci[0])

        blk_v = (blk_a, blk_b)
        pack_v = (pack_a, pack_b)
        sem_pi = (sem_pia, sem_pib)
        sem_po = (sem_poa, sem_pob)

        def make_pack(blk_ref, pack_ref):
            def pack_g(gg):
                bb = gg >> 3
                goff = (gg & 7) * _LANES
                rows = bb * 128 + goff + lanes
                for c in range(8):
                    vec = blk_ref[bb * 8 + c, pl.ds(goff, _LANES)]
                    plsc.store_scatter(pack_ref, [rows, _full(c)], vec)
            return pack_g

        def pack_start(j, b):
            iter_start = sid * BLK_PER_TILE + j * PACK_BLKS
            return pltpu.async_copy(
                view_hbm.at[pl.ds(iter_start * 8, 8 * PACK_BLKS)],
                blk_v[b], sem_pi[b])

        pins = [pack_start(0, 0), None]
        pouts = [None, None]
        for j in range(PACK_IT):
            b = j % 2
            nb = 1 - b
            if j + 1 < PACK_IT:
                pins[nb] = pack_start(j + 1, nb)
            pins[b].wait()
            if j >= 2:
                pouts[b].wait()
            plsc.parallel_loop(0, 8 * PACK_BLKS, unroll=2)(make_pack(blk_v[b], pack_v[b]))
            iter_start = sid * BLK_PER_TILE + j * PACK_BLKS
            pouts[b] = pltpu.async_copy(
                pack_v[b],
                scr_hbm.at[pl.ds(cid * FP + iter_start * 128, PACK_FR)],
                sem_po[b])
        pouts[0].wait()
        pouts[1].wait()
        plsc.subcore_barrier()

        def make_init(out_ref):
            def init_g(g):
                base = g * _LANES
                off = 3 * (4 * CH) + ((base >> 7) << 9) + (base & 127)
                out_ref[pl.ds(off, _LANES)] = zero
                out_ref[pl.ds(off + 128, _LANES)] = zero
                out_ref[pl.ds(off + 256, _LANES)] = zero
                out_ref[pl.ds(off + 384, _LANES)] = one
            return init_g

        plsc.parallel_loop(0, G)(make_init(out_v[0]))
        plsc.parallel_loop(0, G)(make_init(out_v[1]))

        def make_shift(ri_ref):
            def shift_g(g):
                b2 = g * _LANES
                ri_ref[pl.ds(b2, _LANES)] = ri_ref[pl.ds(b2, _LANES)] + FP
            return shift_g

        def make_compute(ci_ref, rows_ref, out_ref):
            def compute_g(g):
                base = g * _LANES
                rid = base + lanes
                ci16 = ci_ref[pl.ds(base, _LANES)]

                def camg(col):
                    return plsc.load_gather(cam_v, [ci16, _full(col)])

                cw, cx, cy, cz = camg(0), camg(1), camg(2), camg(3)
                tcx, tcy, tcz = camg(4), camg(5), camg(6)

                def rigg(col):
                    return plsc.load_gather(rows_ref, [rid, _full(col)])

                rw, rx, ry, rz = rigg(0), rigg(1), rigg(2), rigg(3)
                tx, ty, tz = rigg(4), rigg(5), rigg(6)

                w = cw * rw - cx * rx - cy * ry - cz * rz
                x = cw * rx + cx * rw + cy * rz - cz * ry
                y = cw * ry - cx * rz + cy * rw + cz * rx
                z = cw * rz + cx * ry - cy * rx + cz * rw

                x2, y2, z2 = x + x, y + y, z + z
                xx, yy, zz = x * x2, y * y2, z * z2
                xy, xz, yz = x * y2, x * z2, y * z2
                wx, wy, wz = w * x2, w * y2, w * z2
                r00 = 1.0 - (yy + zz)
                r01 = xy - wz
                r02 = xz + wy
                r10 = xy + wz
                r11 = 1.0 - (xx + zz)
                r12 = yz - wx
                r20 = xz - wy
                r21 = yz + wx
                r22 = 1.0 - (xx + yy)

                ux = cy * tz - cz * ty
                uy = cz * tx - cx * tz
                uz = cx * ty - cy * tx
                vx = ux + cw * tx
                vy = uy + cw * ty
                vz = uz + cw * tz
                px = cy * vz - cz * vy
                py = cz * vx - cx * vz
                pz = cx * vy - cy * vx
                ttx = tx + (px + px) + tcx
                tty = ty + (py + py) + tcy
                ttz = tz + (pz + pz) + tcz

                outs = (r00, r01, r02, ttx,
                        r10, r11, r12, tty,
                        r20, r21, r22, ttz)
                common = ((base >> 7) << 9) + (base & 127)
                for col, val in enumerate(outs):
                    r, c = divmod(col, 4)
                    out_ref[pl.ds(r * (4 * CH) + common + c * 128,
                                  _LANES)] = val
            return compute_g

        ri0.wait()

        @pl.when(cid == 1)
        def _():
            plsc.parallel_loop(0, G)(make_shift(ri_v[0]))

        gathers = [None, None]
        gathers[0] = pltpu.async_copy(scr_hbm.at[ri_v[0]], rows_v[0],
                                      sem_g[0])
        ci_copies = [ci0, None]
        out_copies = [None, None]

        for k in range(NCHUNK):
            b = k % 2
            nb = 1 - b
            if k + 1 < NCHUNK:
                nbase = wid * BPW + (k + 1) * CH
                rin = pltpu.async_copy(ri_hbm.at[pl.ds(nbase, CH)],
                                       ri_v[nb], sem_ri[nb])
                ci_copies[nb] = pltpu.async_copy(
                    ci_hbm.at[pl.ds(nbase, CH)], ci_v[nb], sem_ci[nb])
            gathers[b].wait()
            ci_copies[b].wait()
            if k + 1 < NCHUNK:
                rin.wait()

                @pl.when(cid == 1)
                def _():
                    plsc.parallel_loop(0, G)(make_shift(ri_v[nb]))

                gathers[nb] = pltpu.async_copy(scr_hbm.at[ri_v[nb]],
                                               rows_v[nb], sem_g[nb])
            if k >= 2:
                for h in out_copies[b]:
                    h.wait()
            plsc.parallel_loop(0, G, unroll=2)(
                make_compute(ci_v[b], rows_v[b], out_v[b]))
            base = wid * BPW + k * CH
            out_copies[b] = [
                pltpu.async_copy(
                    out_v[b].at[pl.ds(r * (4 * CH), 4 * CH)],
                    out_hbm.at[pl.ds(r * (4 * B) + base * 4, 4 * CH)],
                    sem_o[b])
                for r in range(4)]
        for b in range(2):
            for h in out_copies[b]:
                h.wait()

    return sc_fn


def kernel(q_cam, t_cam, q_rig, t_rig, camera_index, rig_index):
    B = camera_index.shape[0]
    F = q_rig.shape[0]
    NCAM = q_cam.shape[0]
    FP = 16 * 50 * 128
    cam_packed = jnp.concatenate(
        [q_cam.astype(jnp.float32), t_cam.astype(jnp.float32),
         jnp.zeros((NCAM, 1), jnp.float32)], axis=1)
    ci = camera_index.astype(jnp.int32)
    ri = rig_index.astype(jnp.int32)
    rig_view = (jnp.concatenate(
        [jnp.pad(q_rig.astype(jnp.float32), ((0, FP - F), (0, 0))),
         jnp.pad(t_rig.astype(jnp.float32), ((0, FP - F), (0, 1)))], axis=1)
                .reshape(FP // 128, 128, 8)
                .transpose(0, 2, 1)
                .reshape(FP * 8 // 128, 128))
    out, _ = _make_sc_compose(B, F, NCAM)(cam_packed, rig_view, ci, ri)
    return (out.reshape(4, B // 128, 4, 128)
            .transpose(1, 3, 0, 2)
            .reshape(B, 4, 4))
